# speedup vs baseline: 6.2040x; 6.2040x over previous
import functools

import jax
import jax.numpy as jnp
from jax import lax
from jax.experimental import pallas as pl
from jax.experimental.pallas import tpu as pltpu
from jax.experimental.pallas import tpu_sc as plsc

HIDDEN = 128
CHUNK = 128
NBUF = 4
NC = 2
NS = 16
NW = NC * NS


@functools.partial(jax.jit, static_argnums=(0, 1))
def _lookup(n_tokens, chunks_per_w, idx, table):
    per_w = chunks_per_w * CHUNK
    groups = chunks_per_w // NBUF
    mesh = plsc.VectorSubcoreMesh(core_axis_name="c", subcore_axis_name="s")

    @functools.partial(
        pl.kernel,
        mesh=mesh,
        out_type=jax.ShapeDtypeStruct((n_tokens, HIDDEN), jnp.float32),
        scratch_types=[
            pltpu.VMEM((chunks_per_w, CHUNK), jnp.int32),
            pltpu.VMEM((NBUF, CHUNK, HIDDEN), jnp.float32),
            pltpu.SemaphoreType.DMA,
            pltpu.SemaphoreType.DMA,
        ],
    )
    def k(idx_hbm, table_hbm, out_hbm, idx_v, rows_v, gsem, wsem):
        wid = lax.axis_index("s") * NC + lax.axis_index("c")
        base = wid * per_w
        pltpu.sync_copy(idx_hbm.at[wid], idx_v)

        def group(g, carry):
            gh = [
                pltpu.async_copy(
                    table_hbm.at[idx_v.at[g * NBUF + b]], rows_v.at[b], gsem
                )
                for b in range(NBUF)
            ]
            for h in gh:
                h.wait()
            wh = [
                pltpu.async_copy(
                    rows_v.at[b],
                    out_hbm.at[pl.ds(base + (g * NBUF + b) * CHUNK, CHUNK)],
                    wsem,
                )
                for b in range(NBUF)
            ]
            for h in wh:
                h.wait()
            return carry

        lax.fori_loop(0, groups, group, 0)

    return k(idx, table)


def kernel(batch, doc_len, embed_weight):
    del doc_len
    bsz, seq = batch.shape
    n_tokens = bsz * seq
    chunks_per_w = n_tokens // (NW * CHUNK)
    idx = batch.reshape(NW, chunks_per_w, CHUNK).astype(jnp.int32)
    out = _lookup(n_tokens, chunks_per_w, idx, embed_weight)
    return out.reshape(bsz, seq, HIDDEN)
